# baseline (speedup 1.0000x reference)
import functools

import jax
import jax.numpy as jnp
from jax import lax
from jax.experimental import pallas as pl
from jax.experimental.pallas import tpu as pltpu
from jax.experimental.pallas import tpu_sc as plsc

_RG = 16
_MC = 1024
_N_SC = 2048
_NSUB = 16
_SCL = 32


def _tc_body(
    x_ref, y_ref, rsum_ref, cmin_ref, yt_ref, ymin_ref, rmin_ref,
    *, nt_steps, tn, m
):
    b = pl.program_id(0)
    nt = pl.program_id(1)
    inf = jnp.array(float("inf"), jnp.bfloat16)

    @pl.when(jnp.logical_and(b == 0, nt == 0))
    def _init_loss():
        rsum_ref[0, 0] = 0.0

    @pl.when(nt == 0)
    def _prep_y():
        yt_ref[...] = jnp.transpose(y_ref[0]).astype(jnp.bfloat16)
        ymin_ref[...] = jnp.full((_RG, m), inf, jnp.bfloat16)

    x = x_ref[0].astype(jnp.bfloat16)

    for mc in range(m // _MC):
        sl = slice(mc * _MC, (mc + 1) * _MC)
        y0 = yt_ref[0:1, sl]
        y1 = yt_ref[1:2, sl]
        y2 = yt_ref[2:3, sl]
        ym_acc = None
        for rg in range(tn // _RG):
            rs = slice(rg * _RG, (rg + 1) * _RG)
            xr = x[rs, :]
            d = (
                jnp.abs(xr[:, 0:1] - y0)
                + jnp.abs(xr[:, 1:2] - y1)
                + jnp.abs(xr[:, 2:3] - y2)
            )
            ym_acc = d if ym_acc is None else jnp.minimum(ym_acc, d)
            parts = [d[:, k * 128:(k + 1) * 128] for k in range(_MC // 128)]
            while len(parts) > 1:
                parts = [
                    jnp.minimum(parts[i], parts[i + 1])
                    for i in range(0, len(parts) - 1, 2)
                ] + ([parts[-1]] if len(parts) % 2 else [])
            dm = parts[0]
            if mc == 0:
                rmin_ref[rs, :] = dm
            else:
                rmin_ref[rs, :] = jnp.minimum(rmin_ref[rs, :], dm)
        ymin_ref[:, sl] = jnp.minimum(ymin_ref[:, sl], ym_acc)

    sx = jnp.sum(jnp.min(rmin_ref[...], axis=1).astype(jnp.float32))
    rsum_ref[0, 0] += sx

    @pl.when(nt == nt_steps - 1)
    def _emit_cmin():
        cmin_ref[0] = ymin_ref[...]


def _tc_main(x_tc, mesh_y, tn):
    B, n_tc, D = x_tc.shape
    _, M, _ = mesh_y.shape
    nt = n_tc // tn
    body = functools.partial(_tc_body, nt_steps=nt, tn=tn, m=M)
    return pl.pallas_call(
        body,
        grid=(B, nt),
        in_specs=[
            pl.BlockSpec((1, tn, D), lambda b, i: (b, i, 0)),
            pl.BlockSpec((1, M, D), lambda b, i: (b, 0, 0)),
        ],
        out_specs=[
            pl.BlockSpec((1, 1), lambda b, i: (0, 0), memory_space=pltpu.SMEM),
            pl.BlockSpec((1, _RG, M), lambda b, i: (b, 0, 0)),
        ],
        out_shape=[
            jax.ShapeDtypeStruct((1, 1), jnp.float32),
            jax.ShapeDtypeStruct((B, _RG, M), jnp.bfloat16),
        ],
        scratch_shapes=[
            pltpu.VMEM((D, M), jnp.bfloat16),
            pltpu.VMEM((_RG, M), jnp.bfloat16),
            pltpu.VMEM((tn, 128), jnp.bfloat16),
        ],
    )(x_tc, mesh_y)


def _sc_make(B, M, rows_w):
    mesh = plsc.VectorSubcoreMesh(core_axis_name="c", subcore_axis_name="s")
    RB = 8

    @functools.partial(
        pl.kernel,
        out_type=(
            jax.ShapeDtypeStruct((B, _NSUB, 1, rows_w * _SCL), jnp.bfloat16),
            jax.ShapeDtypeStruct((B, _NSUB, 1, M), jnp.bfloat16),
        ),
        mesh=mesh,
        scratch_types=[
            pltpu.VMEM((3, rows_w, _SCL), jnp.bfloat16),
            pltpu.VMEM((3, M), jnp.bfloat16),
            pltpu.VMEM((1, M), jnp.bfloat16),
            pltpu.VMEM((1, rows_w * _SCL), jnp.bfloat16),
        ],
    )
    def sc_kernel(xp_hbm, yt_hbm, rmin_hbm, cmin_hbm, xvb, yv, cminv, rminv):
        c = lax.axis_index("c")
        s = lax.axis_index("s")
        pltpu.sync_copy(xp_hbm.at[c, s], xvb)
        pltpu.sync_copy(yt_hbm.at[c], yv)

        inf_v = jnp.full((_SCL,), float("inf"), jnp.bfloat16)
        for j in range(M // _SCL):
            cminv[0, pl.ds(j * _SCL, _SCL)] = inf_v

        for blk in range(rows_w // RB):
            xbs = [
                [xvb[k, blk * RB + l, :] for k in range(3)]
                for l in range(RB)
            ]

            def jbody(j, carry, xbs=xbs):
                sl = pl.ds(j * _SCL, _SCL)
                y0 = yv[0, sl]
                y1 = yv[1, sl]
                y2 = yv[2, sl]
                cm = cminv[0, sl]
                rmins = list(carry)
                for l in range(RB):
                    d = (
                        jnp.abs(xbs[l][0] - y0)
                        + jnp.abs(xbs[l][1] - y1)
                        + jnp.abs(xbs[l][2] - y2)
                    )
                    cm = jnp.minimum(cm, d)
                    rmins[l] = jnp.minimum(rmins[l], d)
                cminv[0, sl] = cm
                return tuple(rmins)

            rmins = lax.fori_loop(0, M // _SCL, jbody, (inf_v,) * RB)
            for l in range(RB):
                rminv[0, pl.ds((blk * RB + l) * _SCL, _SCL)] = rmins[l]

        pltpu.sync_copy(rminv, rmin_hbm.at[c, s])
        pltpu.sync_copy(cminv, cmin_hbm.at[c, s])

    return sc_kernel


def _epi_body(trs_ref, tcm_ref, srm_ref, scm_ref, out_ref, *, n, m, b_total):
    total_r = trs_ref[0, 0] + jnp.sum(
        jnp.min(srm_ref[...], axis=1).astype(jnp.float32)
    )
    colsum = jnp.float32(0.0)
    for b in range(b_total):
        cb = jnp.minimum(
            jnp.min(scm_ref[b], axis=0),
            jnp.min(tcm_ref[b], axis=0),
        )
        colsum += jnp.sum(cb.astype(jnp.float32))
    out_ref[0, 0] = total_r / (n * b_total) + colsum / (m * b_total)


def _epilogue(t_rsum, t_cmin, s_rsum, s_cmin, n, m):
    B = t_cmin.shape[0]
    body = functools.partial(_epi_body, n=float(n), m=float(m), b_total=B)
    return pl.pallas_call(
        body,
        in_specs=[
            pl.BlockSpec(memory_space=pltpu.SMEM),
            pl.BlockSpec(),
            pl.BlockSpec(),
            pl.BlockSpec(),
        ],
        out_specs=pl.BlockSpec(memory_space=pltpu.SMEM),
        out_shape=jax.ShapeDtypeStruct((1, 1), jnp.float32),
    )(t_rsum, t_cmin, s_rsum, s_cmin)


def kernel(mesh_x, mesh_y):
    B, N, D = mesh_x.shape
    _, M, _ = mesh_y.shape
    n_sc = _N_SC
    n_tc = N - n_sc
    rows_w = n_sc // _NSUB

    x_tc = mesh_x[:, :n_tc, :]
    xp = jnp.broadcast_to(
        jnp.transpose(
            jnp.transpose(mesh_x[:, n_tc:, :], (0, 2, 1))
            .astype(jnp.bfloat16)
            .reshape(B, D, _NSUB, rows_w),
            (0, 2, 1, 3),
        )[..., None],
        (B, _NSUB, D, rows_w, _SCL),
    )
    yt_bf = jnp.transpose(mesh_y, (0, 2, 1)).astype(jnp.bfloat16)

    s_rmin, s_cmin = _sc_make(B, M, rows_w)(xp, yt_bf)
    t_rsum, t_cmin = _tc_main(x_tc, mesh_y, tn=n_tc // 1)
    loss = _epilogue(
        t_rsum,
        t_cmin,
        s_rmin.reshape(B * _NSUB * rows_w, _SCL),
        s_cmin.reshape(B, _NSUB, M),
        N,
        M,
    )
    return loss[0, 0]


# speedup vs baseline: 1.4756x; 1.4756x over previous
import functools

import jax
import jax.numpy as jnp
from jax import lax
from jax.experimental import pallas as pl
from jax.experimental.pallas import tpu as pltpu
from jax.experimental.pallas import tpu_sc as plsc

_RG = 16
_MC = 1024
_N_SC = 1024
_NSUB = 16
_SCL = 32


def _tc_body(
    x_ref, y_ref, rsum_ref, cmin_ref, yt_ref, ymin_ref, rmin_ref,
    *, nt_steps, tn, m
):
    b = pl.program_id(0)
    nt = pl.program_id(1)
    inf = jnp.array(float("inf"), jnp.bfloat16)

    @pl.when(jnp.logical_and(b == 0, nt == 0))
    def _init_loss():
        rsum_ref[0, 0] = 0.0

    @pl.when(nt == 0)
    def _prep_y():
        yt_ref[...] = jnp.transpose(y_ref[0]).astype(jnp.bfloat16)
        ymin_ref[...] = jnp.full((_RG, m), inf, jnp.bfloat16)

    x = x_ref[0].astype(jnp.bfloat16)

    for mc in range(m // _MC):
        sl = slice(mc * _MC, (mc + 1) * _MC)
        y0 = yt_ref[0:1, sl]
        y1 = yt_ref[1:2, sl]
        y2 = yt_ref[2:3, sl]
        ym_acc = None
        for rg in range(tn // _RG):
            rs = slice(rg * _RG, (rg + 1) * _RG)
            xr = x[rs, :]
            d = (
                jnp.abs(xr[:, 0:1] - y0)
                + jnp.abs(xr[:, 1:2] - y1)
                + jnp.abs(xr[:, 2:3] - y2)
            )
            ym_acc = d if ym_acc is None else jnp.minimum(ym_acc, d)
            parts = [d[:, k * 128:(k + 1) * 128] for k in range(_MC // 128)]
            while len(parts) > 1:
                parts = [
                    jnp.minimum(parts[i], parts[i + 1])
                    for i in range(0, len(parts) - 1, 2)
                ] + ([parts[-1]] if len(parts) % 2 else [])
            dm = parts[0]
            if mc == 0:
                rmin_ref[rs, :] = dm
            else:
                rmin_ref[rs, :] = jnp.minimum(rmin_ref[rs, :], dm)
        ymin_ref[:, sl] = jnp.minimum(ymin_ref[:, sl], ym_acc)

    sx = jnp.sum(jnp.min(rmin_ref[...], axis=1).astype(jnp.float32))
    rsum_ref[0, 0] += sx

    @pl.when(nt == nt_steps - 1)
    def _emit_cmin():
        cmin_ref[0] = ymin_ref[...]


def _tc_main(x_tc, mesh_y, tn):
    B, n_tc, D = x_tc.shape
    _, M, _ = mesh_y.shape
    nt = n_tc // tn
    body = functools.partial(_tc_body, nt_steps=nt, tn=tn, m=M)
    return pl.pallas_call(
        body,
        grid=(B, nt),
        in_specs=[
            pl.BlockSpec((1, tn, D), lambda b, i: (b, i, 0)),
            pl.BlockSpec((1, M, D), lambda b, i: (b, 0, 0)),
        ],
        out_specs=[
            pl.BlockSpec((1, 1), lambda b, i: (0, 0), memory_space=pltpu.SMEM),
            pl.BlockSpec((1, _RG, M), lambda b, i: (b, 0, 0)),
        ],
        out_shape=[
            jax.ShapeDtypeStruct((1, 1), jnp.float32),
            jax.ShapeDtypeStruct((B, _RG, M), jnp.bfloat16),
        ],
        scratch_shapes=[
            pltpu.VMEM((D, M), jnp.bfloat16),
            pltpu.VMEM((_RG, M), jnp.bfloat16),
            pltpu.VMEM((tn, 128), jnp.bfloat16),
        ],
    )(x_tc, mesh_y)


def _sc_make(B, M, rows_w):
    mesh = plsc.VectorSubcoreMesh(core_axis_name="c", subcore_axis_name="s")
    RB = 8

    @functools.partial(
        pl.kernel,
        out_type=(
            jax.ShapeDtypeStruct((B, _NSUB, 1, rows_w * _SCL), jnp.bfloat16),
            jax.ShapeDtypeStruct((B, _NSUB, 1, M), jnp.bfloat16),
        ),
        mesh=mesh,
        scratch_types=[
            pltpu.VMEM((3, rows_w, _SCL), jnp.bfloat16),
            pltpu.VMEM((3, M), jnp.bfloat16),
            pltpu.VMEM((1, M), jnp.bfloat16),
            pltpu.VMEM((1, rows_w * _SCL), jnp.bfloat16),
        ],
    )
    def sc_kernel(xp_hbm, yt_hbm, rmin_hbm, cmin_hbm, xvb, yv, cminv, rminv):
        c = lax.axis_index("c")
        s = lax.axis_index("s")
        pltpu.sync_copy(xp_hbm.at[c, s], xvb)
        pltpu.sync_copy(yt_hbm.at[c], yv)

        inf_v = jnp.full((_SCL,), float("inf"), jnp.bfloat16)
        for j in range(M // _SCL):
            cminv[0, pl.ds(j * _SCL, _SCL)] = inf_v

        for blk in range(rows_w // RB):
            xbs = [
                [xvb[k, blk * RB + l, :] for k in range(3)]
                for l in range(RB)
            ]

            def jbody(j, carry, xbs=xbs):
                sl = pl.ds(j * _SCL, _SCL)
                y0 = yv[0, sl]
                y1 = yv[1, sl]
                y2 = yv[2, sl]
                cm = cminv[0, sl]
                rmins = list(carry)
                for l in range(RB):
                    d = (
                        jnp.abs(xbs[l][0] - y0)
                        + jnp.abs(xbs[l][1] - y1)
                        + jnp.abs(xbs[l][2] - y2)
                    )
                    cm = jnp.minimum(cm, d)
                    rmins[l] = jnp.minimum(rmins[l], d)
                cminv[0, sl] = cm
                return tuple(rmins)

            rmins = lax.fori_loop(0, M // _SCL, jbody, (inf_v,) * RB)
            for l in range(RB):
                rminv[0, pl.ds((blk * RB + l) * _SCL, _SCL)] = rmins[l]

        pltpu.sync_copy(rminv, rmin_hbm.at[c, s])
        pltpu.sync_copy(cminv, cmin_hbm.at[c, s])

    return sc_kernel


def _epi_body(trs_ref, tcm_ref, srm_ref, scm_ref, out_ref, *, n, m, b_total):
    total_r = trs_ref[0, 0] + jnp.sum(
        jnp.min(srm_ref[...], axis=1).astype(jnp.float32)
    )
    colsum = jnp.float32(0.0)
    for b in range(b_total):
        cb = jnp.minimum(
            jnp.min(scm_ref[b], axis=0),
            jnp.min(tcm_ref[b], axis=0),
        )
        colsum += jnp.sum(cb.astype(jnp.float32))
    out_ref[0, 0] = total_r / (n * b_total) + colsum / (m * b_total)


def _epilogue(t_rsum, t_cmin, s_rsum, s_cmin, n, m):
    B = t_cmin.shape[0]
    body = functools.partial(_epi_body, n=float(n), m=float(m), b_total=B)
    return pl.pallas_call(
        body,
        in_specs=[
            pl.BlockSpec(memory_space=pltpu.SMEM),
            pl.BlockSpec(),
            pl.BlockSpec(),
            pl.BlockSpec(),
        ],
        out_specs=pl.BlockSpec(memory_space=pltpu.SMEM),
        out_shape=jax.ShapeDtypeStruct((1, 1), jnp.float32),
    )(t_rsum, t_cmin, s_rsum, s_cmin)


def kernel(mesh_x, mesh_y):
    B, N, D = mesh_x.shape
    _, M, _ = mesh_y.shape
    n_sc = _N_SC
    n_tc = N - n_sc
    rows_w = n_sc // _NSUB

    x_tc = mesh_x[:, :n_tc, :]
    xp = jnp.broadcast_to(
        jnp.transpose(
            jnp.transpose(mesh_x[:, n_tc:, :], (0, 2, 1))
            .astype(jnp.bfloat16)
            .reshape(B, D, _NSUB, rows_w),
            (0, 2, 1, 3),
        )[..., None],
        (B, _NSUB, D, rows_w, _SCL),
    )
    yt_bf = jnp.transpose(mesh_y, (0, 2, 1)).astype(jnp.bfloat16)

    s_rmin, s_cmin = _sc_make(B, M, rows_w)(xp, yt_bf)
    t_rsum, t_cmin = _tc_main(x_tc, mesh_y, tn=n_tc // 1)
    loss = _epilogue(
        t_rsum,
        t_cmin,
        s_rmin.reshape(B * _NSUB * rows_w, _SCL),
        s_cmin.reshape(B, _NSUB, M),
        N,
        M,
    )
    return loss[0, 0]


# speedup vs baseline: 1.6123x; 1.0927x over previous
import functools

import jax
import jax.numpy as jnp
from jax import lax
from jax.experimental import pallas as pl
from jax.experimental.pallas import tpu as pltpu
from jax.experimental.pallas import tpu_sc as plsc

_RG = 16
_MC = 1024
_N_SC = 768
_NSUB = 16
_SCL = 32


def _tc_body(
    x_ref, y_ref, rsum_ref, cmin_ref, yt_ref, ymin_ref, rmin_ref,
    *, nt_steps, tn, m
):
    b = pl.program_id(0)
    nt = pl.program_id(1)
    inf = jnp.array(float("inf"), jnp.bfloat16)

    @pl.when(jnp.logical_and(b == 0, nt == 0))
    def _init_loss():
        rsum_ref[0, 0] = 0.0

    @pl.when(nt == 0)
    def _prep_y():
        yt_ref[...] = jnp.transpose(y_ref[0]).astype(jnp.bfloat16)
        ymin_ref[...] = jnp.full((_RG, m), inf, jnp.bfloat16)

    x = x_ref[0].astype(jnp.bfloat16)

    for mc in range(m // _MC):
        sl = slice(mc * _MC, (mc + 1) * _MC)
        y0 = yt_ref[0:1, sl]
        y1 = yt_ref[1:2, sl]
        y2 = yt_ref[2:3, sl]
        ym_acc = None
        for rg in range(tn // _RG):
            rs = slice(rg * _RG, (rg + 1) * _RG)
            xr = x[rs, :]
            d = (
                jnp.abs(xr[:, 0:1] - y0)
                + jnp.abs(xr[:, 1:2] - y1)
                + jnp.abs(xr[:, 2:3] - y2)
            )
            ym_acc = d if ym_acc is None else jnp.minimum(ym_acc, d)
            parts = [d[:, k * 128:(k + 1) * 128] for k in range(_MC // 128)]
            while len(parts) > 1:
                parts = [
                    jnp.minimum(parts[i], parts[i + 1])
                    for i in range(0, len(parts) - 1, 2)
                ] + ([parts[-1]] if len(parts) % 2 else [])
            dm = parts[0]
            if mc == 0:
                rmin_ref[rs, :] = dm
            else:
                rmin_ref[rs, :] = jnp.minimum(rmin_ref[rs, :], dm)
        ymin_ref[:, sl] = jnp.minimum(ymin_ref[:, sl], ym_acc)

    sx = jnp.sum(jnp.min(rmin_ref[...], axis=1).astype(jnp.float32))
    rsum_ref[0, 0] += sx

    @pl.when(nt == nt_steps - 1)
    def _emit_cmin():
        cmin_ref[0] = ymin_ref[...]


def _tc_main(mesh_x, mesh_y, n_tc, tn):
    B, _, D = mesh_x.shape
    _, M, _ = mesh_y.shape
    nt = n_tc // tn
    body = functools.partial(_tc_body, nt_steps=nt, tn=tn, m=M)
    return pl.pallas_call(
        body,
        grid=(B, nt),
        in_specs=[
            pl.BlockSpec((1, tn, D), lambda b, i: (b, i, 0)),
            pl.BlockSpec((1, M, D), lambda b, i: (b, 0, 0)),
        ],
        out_specs=[
            pl.BlockSpec((1, 1), lambda b, i: (0, 0), memory_space=pltpu.SMEM),
            pl.BlockSpec((1, _RG, M), lambda b, i: (b, 0, 0)),
        ],
        out_shape=[
            jax.ShapeDtypeStruct((1, 1), jnp.float32),
            jax.ShapeDtypeStruct((B, _RG, M), jnp.bfloat16),
        ],
        scratch_shapes=[
            pltpu.VMEM((D, M), jnp.bfloat16),
            pltpu.VMEM((_RG, M), jnp.bfloat16),
            pltpu.VMEM((tn, 128), jnp.bfloat16),
        ],
    )(mesh_x, mesh_y)


def _sc_make(B, M, rows_w):
    mesh = plsc.VectorSubcoreMesh(core_axis_name="c", subcore_axis_name="s")
    RB = 8

    @functools.partial(
        pl.kernel,
        out_type=(
            jax.ShapeDtypeStruct((B, _NSUB, 1, rows_w * _SCL), jnp.bfloat16),
            jax.ShapeDtypeStruct((B, _NSUB, 1, M), jnp.bfloat16),
        ),
        mesh=mesh,
        scratch_types=[
            pltpu.VMEM((3, rows_w, _SCL), jnp.bfloat16),
            pltpu.VMEM((3, M), jnp.bfloat16),
            pltpu.VMEM((1, M), jnp.bfloat16),
            pltpu.VMEM((1, rows_w * _SCL), jnp.bfloat16),
        ],
    )
    def sc_kernel(xp_hbm, yt_hbm, rmin_hbm, cmin_hbm, xvb, yv, cminv, rminv):
        c = lax.axis_index("c")
        s = lax.axis_index("s")
        pltpu.sync_copy(xp_hbm.at[c, s], xvb)
        pltpu.sync_copy(yt_hbm.at[c], yv)

        inf_v = jnp.full((_SCL,), float("inf"), jnp.bfloat16)
        for j in range(M // _SCL):
            cminv[0, pl.ds(j * _SCL, _SCL)] = inf_v

        for blk in range(rows_w // RB):
            xbs = [
                [xvb[k, blk * RB + l, :] for k in range(3)]
                for l in range(RB)
            ]

            def jbody(j, carry, xbs=xbs):
                sl = pl.ds(j * _SCL, _SCL)
                y0 = yv[0, sl]
                y1 = yv[1, sl]
                y2 = yv[2, sl]
                cm = cminv[0, sl]
                rmins = list(carry)
                for l in range(RB):
                    d = (
                        jnp.abs(xbs[l][0] - y0)
                        + jnp.abs(xbs[l][1] - y1)
                        + jnp.abs(xbs[l][2] - y2)
                    )
                    cm = jnp.minimum(cm, d)
                    rmins[l] = jnp.minimum(rmins[l], d)
                cminv[0, sl] = cm
                return tuple(rmins)

            rmins = lax.fori_loop(0, M // _SCL, jbody, (inf_v,) * RB)
            for l in range(RB):
                rminv[0, pl.ds((blk * RB + l) * _SCL, _SCL)] = rmins[l]

        pltpu.sync_copy(rminv, rmin_hbm.at[c, s])
        pltpu.sync_copy(cminv, cmin_hbm.at[c, s])

    return sc_kernel


def _epi_body(trs_ref, tcm_ref, srm_ref, scm_ref, out_ref, *, n, m, b_total):
    total_r = trs_ref[0, 0] + jnp.sum(
        jnp.min(srm_ref[...], axis=-1).astype(jnp.float32)
    )
    colsum = jnp.float32(0.0)
    for b in range(b_total):
        cb = jnp.minimum(
            jnp.min(scm_ref[b], axis=0),
            jnp.min(tcm_ref[b], axis=0),
        )
        colsum += jnp.sum(cb.astype(jnp.float32))
    out_ref[0, 0] = total_r / (n * b_total) + colsum / (m * b_total)


def _epilogue(t_rsum, t_cmin, s_rsum, s_cmin, n, m):
    B = t_cmin.shape[0]
    body = functools.partial(_epi_body, n=float(n), m=float(m), b_total=B)
    return pl.pallas_call(
        body,
        in_specs=[
            pl.BlockSpec(memory_space=pltpu.SMEM),
            pl.BlockSpec(),
            pl.BlockSpec(),
            pl.BlockSpec(),
        ],
        out_specs=pl.BlockSpec(memory_space=pltpu.SMEM),
        out_shape=jax.ShapeDtypeStruct((1, 1), jnp.float32),
    )(t_rsum, t_cmin, s_rsum, s_cmin)


def kernel(mesh_x, mesh_y):
    B, N, D = mesh_x.shape
    _, M, _ = mesh_y.shape
    n_sc = _N_SC
    n_tc = N - n_sc
    rows_w = n_sc // _NSUB

    xp = jnp.broadcast_to(
        jnp.transpose(
            jnp.transpose(mesh_x[:, n_tc:, :], (0, 2, 1))
            .astype(jnp.bfloat16)
            .reshape(B, D, _NSUB, rows_w),
            (0, 2, 1, 3),
        )[..., None],
        (B, _NSUB, D, rows_w, _SCL),
    )
    yt_bf = jnp.transpose(mesh_y, (0, 2, 1)).astype(jnp.bfloat16)

    s_rmin, s_cmin = _sc_make(B, M, rows_w)(xp, yt_bf)
    t_rsum, t_cmin = _tc_main(mesh_x, mesh_y, n_tc, tn=n_tc)
    loss = _epilogue(
        t_rsum,
        t_cmin,
        s_rmin.reshape(B, _NSUB, rows_w, _SCL),
        s_cmin.reshape(B, _NSUB, M),
        N,
        M,
    )
    return loss[0, 0]


# speedup vs baseline: 2.5982x; 1.6115x over previous
import functools

import jax
import jax.numpy as jnp
from jax.experimental import pallas as pl
from jax.experimental.pallas import tpu as pltpu

_RG = 16
_MC = 1024


def _chamfer_body(
    x_ref, y_ref, loss_ref, yt_ref, ymin_ref, rmin_ref,
    *, n_total, m_total, nt_steps, b_total, tn, m
):
    b = pl.program_id(0)
    nt = pl.program_id(1)
    inf = jnp.array(float("inf"), jnp.bfloat16)

    @pl.when(jnp.logical_and(b == 0, nt == 0))
    def _init_loss():
        loss_ref[0, 0] = 0.0

    @pl.when(nt == 0)
    def _prep_y():
        yt_ref[...] = jnp.transpose(y_ref[0]).astype(jnp.bfloat16)
        ymin_ref[...] = jnp.full((_RG, m), inf, jnp.bfloat16)

    x = x_ref[0].astype(jnp.bfloat16)

    for mc in range(m // _MC):
        sl = slice(mc * _MC, (mc + 1) * _MC)
        y0 = yt_ref[0:1, sl]
        y1 = yt_ref[1:2, sl]
        y2 = yt_ref[2:3, sl]
        ym_acc = None
        for rg in range(tn // _RG):
            rs = slice(rg * _RG, (rg + 1) * _RG)
            xr = x[rs, :]
            d = (
                jnp.abs(xr[:, 0:1] - y0)
                + jnp.abs(xr[:, 1:2] - y1)
                + jnp.abs(xr[:, 2:3] - y2)
            )
            ym_acc = d if ym_acc is None else jnp.minimum(ym_acc, d)
            parts = [d[:, k * 128:(k + 1) * 128] for k in range(_MC // 128)]
            while len(parts) > 1:
                parts = [
                    jnp.minimum(parts[i], parts[i + 1])
                    for i in range(0, len(parts) - 1, 2)
                ] + ([parts[-1]] if len(parts) % 2 else [])
            dm = parts[0]
            if mc == 0:
                rmin_ref[rs, :] = dm
            else:
                rmin_ref[rs, :] = jnp.minimum(rmin_ref[rs, :], dm)
        ymin_ref[:, sl] = jnp.minimum(ymin_ref[:, sl], ym_acc)

    sx = jnp.sum(jnp.min(rmin_ref[...], axis=1).astype(jnp.float32))
    loss_ref[0, 0] += sx / (n_total * b_total)

    @pl.when(nt == nt_steps - 1)
    def _finish_batch():
        ys = jnp.sum(jnp.min(ymin_ref[...], axis=0).astype(jnp.float32))
        loss_ref[0, 0] += ys / (m_total * b_total)


def kernel(mesh_x, mesh_y):
    B, N, D = mesh_x.shape
    _, M, _ = mesh_y.shape
    TN = 2048
    NT = N // TN

    body = functools.partial(
        _chamfer_body,
        n_total=float(N),
        m_total=float(M),
        nt_steps=NT,
        b_total=float(B),
        tn=TN,
        m=M,
    )

    loss = pl.pallas_call(
        body,
        grid=(B, NT),
        in_specs=[
            pl.BlockSpec((1, TN, D), lambda b, nt: (b, nt, 0)),
            pl.BlockSpec((1, M, D), lambda b, nt: (b, 0, 0)),
        ],
        out_specs=pl.BlockSpec(
            (1, 1), lambda b, nt: (0, 0), memory_space=pltpu.SMEM
        ),
        out_shape=jax.ShapeDtypeStruct((1, 1), jnp.float32),
        scratch_shapes=[
            pltpu.VMEM((D, M), jnp.bfloat16),
            pltpu.VMEM((_RG, M), jnp.bfloat16),
            pltpu.VMEM((TN, 128), jnp.bfloat16),
        ],
    )(mesh_x, mesh_y)

    return loss[0, 0]


# speedup vs baseline: 2.6994x; 1.0390x over previous
import functools

import jax
import jax.numpy as jnp
from jax.experimental import pallas as pl
from jax.experimental.pallas import tpu as pltpu

_RG = 16
_MC = 1024


def _chamfer_body(
    x_ref, y_ref, loss_ref, yt_ref, ymin_ref, rmin_ref,
    *, n_total, m_total, nt_steps, b_total, tn, m
):
    b = pl.program_id(0)
    nt = pl.program_id(1)
    inf = jnp.array(float("inf"), jnp.bfloat16)

    @pl.when(jnp.logical_and(b == 0, nt == 0))
    def _init_loss():
        loss_ref[0, 0] = 0.0

    @pl.when(nt == 0)
    def _prep_y():
        yt_ref[...] = jnp.transpose(y_ref[0]).astype(jnp.bfloat16)
        ymin_ref[...] = jnp.full((_RG, m), inf, jnp.bfloat16)

    x = x_ref[0].astype(jnp.bfloat16)

    for mc in range(m // _MC):
        sl = slice(mc * _MC, (mc + 1) * _MC)
        y0 = yt_ref[0:1, sl]
        y1 = yt_ref[1:2, sl]
        y2 = yt_ref[2:3, sl]
        ym_acc = None
        for rg in range(tn // _RG):
            rs = slice(rg * _RG, (rg + 1) * _RG)
            xr = x[rs, :]
            d = (
                jnp.abs(xr[:, 0:1] - y0)
                + jnp.abs(xr[:, 1:2] - y1)
                + jnp.abs(xr[:, 2:3] - y2)
            )
            ym_acc = d if ym_acc is None else jnp.minimum(ym_acc, d)
            parts = [d[:, k * 128:(k + 1) * 128] for k in range(_MC // 128)]
            while len(parts) > 1:
                parts = [
                    jnp.minimum(parts[i], parts[i + 1])
                    for i in range(0, len(parts) - 1, 2)
                ] + ([parts[-1]] if len(parts) % 2 else [])
            dm = parts[0]
            if mc == 0:
                rmin_ref[rs, :] = dm
            else:
                rmin_ref[rs, :] = jnp.minimum(rmin_ref[rs, :], dm)
        ymin_ref[:, sl] = jnp.minimum(ymin_ref[:, sl], ym_acc)

    sx = jnp.sum(jnp.min(rmin_ref[...], axis=1).astype(jnp.float32))
    loss_ref[0, 0] += sx / (n_total * b_total)

    @pl.when(nt == nt_steps - 1)
    def _finish_batch():
        ys = jnp.sum(jnp.min(ymin_ref[...], axis=0).astype(jnp.float32))
        loss_ref[0, 0] += ys / (m_total * b_total)


def kernel(mesh_x, mesh_y):
    B, N, D = mesh_x.shape
    _, M, _ = mesh_y.shape
    TN = 4096
    NT = N // TN

    body = functools.partial(
        _chamfer_body,
        n_total=float(N),
        m_total=float(M),
        nt_steps=NT,
        b_total=float(B),
        tn=TN,
        m=M,
    )

    loss = pl.pallas_call(
        body,
        grid=(B, NT),
        in_specs=[
            pl.BlockSpec((1, TN, D), lambda b, nt: (b, nt, 0)),
            pl.BlockSpec((1, M, D), lambda b, nt: (b, 0, 0)),
        ],
        out_specs=pl.BlockSpec(
            (1, 1), lambda b, nt: (0, 0), memory_space=pltpu.SMEM
        ),
        out_shape=jax.ShapeDtypeStruct((1, 1), jnp.float32),
        scratch_shapes=[
            pltpu.VMEM((D, M), jnp.bfloat16),
            pltpu.VMEM((_RG, M), jnp.bfloat16),
            pltpu.VMEM((TN, 128), jnp.bfloat16),
        ],
    )(mesh_x, mesh_y)

    return loss[0, 0]
